# Initial kernel scaffold; baseline (speedup 1.0000x reference)
import jax
import jax.numpy as jnp
from jax.experimental import pallas as pl


def kernel(entity_emb, user_emb, latent_emb, edge_index, edge_type, interact_mat, weight, disen_weight_att):
    raise NotImplementedError("write your pallas kernel here")


# speedup vs baseline: 1.8844x; 1.8844x over previous
import dataclasses

import jax
import jax.numpy as jnp
from jax import lax
from jax.experimental import pallas as pl
from jax.experimental.pallas import tpu as pltpu
from jax.experimental.pallas import tpu_sc as plsc

N_ENTITIES = 10000
CHANNEL = 128
N_EDGES = 320000
N_USERS = 4096
N_REL = 32

NUM_TILES = 32
ENT_PAD = 10240
ROWS_PER_TILE = ENT_PAD // 16
EDGES_PER_TILE = 10240
EDGES_PAD = EDGES_PER_TILE * NUM_TILES
ECHUNK = 64


def _edge_body(ent_hbm, w_hbm, tail_hbm, head_hbm, rel_hbm, psum_hbm, pcnt_hbm,
               tail_v, head_v, rel_v, rows_v, wrows_v, hist_v, acc):
    c = lax.axis_index("c")
    s = lax.axis_index("s")
    wid = s * 2 + c
    ones16 = jnp.ones((16,), jnp.float32)
    zeros16 = jnp.zeros((16,), jnp.float32)

    @pl.loop(0, ECHUNK)
    def _(i):
        for j in range(8):
            rows_v[i, pl.ds(j * 16, 16)] = zeros16

    @pl.loop(0, ENT_PAD, step=16)
    def _(i):
        hist_v[pl.ds(i, 16)] = zeros16

    rbase = s * ROWS_PER_TILE

    @pl.loop(0, ROWS_PER_TILE, step=ECHUNK)
    def _(k):
        pltpu.sync_copy(rows_v, acc.at[pl.ds(rbase + k, ECHUNK)])

    plsc.subcore_barrier()

    ebase = wid * EDGES_PER_TILE

    @pl.loop(0, EDGES_PER_TILE, step=ECHUNK)
    def _(off):
        b = ebase + off
        pltpu.sync_copy(tail_hbm.at[pl.ds(b, ECHUNK)], tail_v)
        pltpu.sync_copy(head_hbm.at[pl.ds(b, ECHUNK)], head_v)
        pltpu.sync_copy(rel_hbm.at[pl.ds(b, ECHUNK)], rel_v)
        pltpu.sync_copy(ent_hbm.at[tail_v], rows_v)
        pltpu.sync_copy(w_hbm.at[rel_v], wrows_v)

        @pl.loop(0, ECHUNK)
        def _(e):
            for j in range(8):
                sl = pl.ds(j * 16, 16)
                rows_v[e, sl] = rows_v[e, sl] * wrows_v[e, sl]

        @pl.loop(0, ECHUNK, step=16)
        def _(e):
            h16 = head_v[pl.ds(e, 16)]
            plsc.addupdate_scatter(hist_v, [h16], ones16)

        pltpu.sync_copy(rows_v, acc.at[head_v], add=True)

    plsc.subcore_barrier()

    @pl.loop(0, ROWS_PER_TILE, step=ECHUNK)
    def _(k):
        pltpu.sync_copy(acc.at[pl.ds(rbase + k, ECHUNK)], rows_v)
        pltpu.sync_copy(rows_v, psum_hbm.at[c, pl.ds(rbase + k, ECHUNK)])

    pltpu.sync_copy(hist_v, pcnt_hbm.at[wid])


def _edge_aggregate(entity_emb, weight, tail_p, head_p, rel_p):
    mesh = plsc.VectorSubcoreMesh(core_axis_name="c", subcore_axis_name="s")
    cp = pltpu.CompilerParams()
    if "needs_layout_passes" in pltpu.CompilerParams.__dataclass_fields__:
        cp = dataclasses.replace(cp, needs_layout_passes=False)
    kern = pl.kernel(
        _edge_body,
        compiler_params=cp,
        out_type=(
            jax.ShapeDtypeStruct((2, ENT_PAD, CHANNEL), jnp.float32),
            jax.ShapeDtypeStruct((NUM_TILES, ENT_PAD), jnp.float32),
        ),
        mesh=mesh,
        scratch_types=[
            pltpu.VMEM((ECHUNK,), jnp.int32),
            pltpu.VMEM((ECHUNK,), jnp.int32),
            pltpu.VMEM((ECHUNK,), jnp.int32),
            pltpu.VMEM((ECHUNK, CHANNEL), jnp.float32),
            pltpu.VMEM((ECHUNK, CHANNEL), jnp.float32),
            pltpu.VMEM((ENT_PAD,), jnp.float32),
            pltpu.VMEM_SHARED((ENT_PAD, CHANNEL), jnp.float32),
        ],
    )
    return kern(entity_emb, weight, tail_p, head_p, rel_p)


def _combine_body(psum_ref, pcnt_ref, out_ref):
    ssum = psum_ref[0] + psum_ref[1]
    cnt_t = jnp.transpose(pcnt_ref[...])
    cnt = jnp.sum(cnt_t, axis=1, keepdims=True)
    out_ref[...] = ssum / jnp.clip(cnt, 1.0, None)


def _combine(psum, pcnt):
    return pl.pallas_call(
        _combine_body,
        out_shape=jax.ShapeDtypeStruct((ENT_PAD, CHANNEL), jnp.float32),
    )(psum, pcnt)


def _user_body(im_ref, ent_ref, ue_ref, lat_ref, dwa_ref, w_ref, out_ref):
    mm = jnp.dot(im_ref[...], ent_ref[...], preferred_element_type=jnp.float32)
    score_ = lax.dot_general(ue_ref[...], lat_ref[...],
                             (((1,), (1,)), ((), ())),
                             preferred_element_type=jnp.float32)
    score = jax.nn.softmax(score_, axis=-1)
    dw = jnp.dot(jax.nn.softmax(dwa_ref[...], axis=-1), w_ref[...],
                 preferred_element_type=jnp.float32)
    gate = jnp.dot(score, dw, preferred_element_type=jnp.float32)
    out_ref[...] = mm * (1.0 + gate)


def _user_aggregate(interact_mat, entity_emb, user_emb, latent_emb,
                    disen_weight_att, weight):
    ub = 256
    grid = (N_USERS // ub,)
    return pl.pallas_call(
        _user_body,
        grid=grid,
        in_specs=[
            pl.BlockSpec((ub, N_ENTITIES), lambda i: (i, 0)),
            pl.BlockSpec((N_ENTITIES, CHANNEL), lambda i: (0, 0)),
            pl.BlockSpec((ub, CHANNEL), lambda i: (i, 0)),
            pl.BlockSpec((4, CHANNEL), lambda i: (0, 0)),
            pl.BlockSpec((4, N_REL), lambda i: (0, 0)),
            pl.BlockSpec((N_REL, CHANNEL), lambda i: (0, 0)),
        ],
        out_specs=pl.BlockSpec((ub, CHANNEL), lambda i: (i, 0)),
        out_shape=jax.ShapeDtypeStruct((N_USERS, CHANNEL), jnp.float32),
    )(interact_mat, entity_emb, user_emb, latent_emb, disen_weight_att, weight)


def kernel(entity_emb, user_emb, latent_emb, edge_index, edge_type,
           interact_mat, weight, disen_weight_att):
    head = edge_index[0].astype(jnp.int32)
    tail = edge_index[1].astype(jnp.int32)
    rel = (edge_type - 1).astype(jnp.int32)

    pad = EDGES_PAD - N_EDGES
    head_p = jnp.concatenate([head, jnp.full((pad,), N_ENTITIES, jnp.int32)])
    tail_p = jnp.concatenate([tail, jnp.zeros((pad,), jnp.int32)])
    rel_p = jnp.concatenate([rel, jnp.zeros((pad,), jnp.int32)])

    psum, pcnt = _edge_aggregate(entity_emb, weight, tail_p, head_p, rel_p)
    entity_agg = _combine(psum, pcnt)[:N_ENTITIES]
    user_agg = _user_aggregate(interact_mat, entity_emb, user_emb, latent_emb,
                               disen_weight_att, weight)
    return (entity_agg, user_agg)


# speedup vs baseline: 2.9267x; 1.5532x over previous
import dataclasses

import jax
import jax.numpy as jnp
from jax import lax
from jax.experimental import pallas as pl
from jax.experimental.pallas import tpu as pltpu
from jax.experimental.pallas import tpu_sc as plsc

N_ENTITIES = 10000
CHANNEL = 128
N_EDGES = 320000
N_USERS = 4096
N_REL = 32

NUM_TILES = 32
ENT_PAD = 10240
ROWS_PER_TILE = ENT_PAD // 16
EDGES_PER_TILE = 10240
EDGES_PAD = EDGES_PER_TILE * NUM_TILES
ECHUNK = 128
CHUNKS_PER_TILE = EDGES_PER_TILE // ECHUNK
TOTAL_CHUNKS = EDGES_PAD // ECHUNK


def _edge_body(ent_hbm, w_hbm, edata_hbm, head_hbm, psum_hbm, pcnt_hbm,
               edata_v, head_v, rows_v, w_local, hist_v, acc):
    c = lax.axis_index("c")
    s = lax.axis_index("s")
    wid = s * 2 + c
    ones16 = jnp.ones((16,), jnp.float32)
    zeros16 = jnp.zeros((16,), jnp.float32)

    pltpu.sync_copy(w_hbm, w_local)

    @pl.loop(0, ECHUNK)
    def _(i):
        for j in range(8):
            rows_v[i, pl.ds(j * 16, 16)] = zeros16

    @pl.loop(0, ENT_PAD, step=16)
    def _(i):
        hist_v[pl.ds(i, 16)] = zeros16

    rbase = s * ROWS_PER_TILE

    @pl.loop(0, ROWS_PER_TILE, step=ECHUNK)
    def _(k):
        pltpu.sync_copy(rows_v, acc.at[pl.ds(rbase + k, ECHUNK)])

    plsc.subcore_barrier()

    cbase = wid * CHUNKS_PER_TILE

    @pl.loop(0, CHUNKS_PER_TILE)
    def _(g):
        chunk = cbase + g
        pltpu.sync_copy(edata_hbm.at[chunk], edata_v)
        pltpu.sync_copy(head_hbm.at[pl.ds(chunk * ECHUNK, ECHUNK)], head_v)
        pltpu.sync_copy(ent_hbm.at[edata_v.at[0]], rows_v)

        @pl.loop(0, ECHUNK, step=16)
        def _(e):
            r16 = edata_v[1, pl.ds(e, 16)]
            for k in range(16):
                r = r16[k]
                for j in range(8):
                    sl = pl.ds(j * 16, 16)
                    rows_v[e + k, sl] = rows_v[e + k, sl] * w_local[r, sl]

        @pl.loop(0, ECHUNK, step=16)
        def _(e):
            h16 = head_v[pl.ds(e, 16)]
            plsc.addupdate_scatter(hist_v, [h16], ones16)

        pltpu.sync_copy(rows_v, acc.at[head_v], add=True)

    plsc.subcore_barrier()

    @pl.loop(0, ROWS_PER_TILE, step=ECHUNK)
    def _(k):
        pltpu.sync_copy(acc.at[pl.ds(rbase + k, ECHUNK)], rows_v)
        pltpu.sync_copy(rows_v, psum_hbm.at[c, pl.ds(rbase + k, ECHUNK)])

    pltpu.sync_copy(hist_v, pcnt_hbm.at[wid])


def _edge_aggregate(entity_emb, weight, edata, head_p):
    mesh = plsc.VectorSubcoreMesh(core_axis_name="c", subcore_axis_name="s")
    cp = pltpu.CompilerParams()
    if "needs_layout_passes" in pltpu.CompilerParams.__dataclass_fields__:
        cp = dataclasses.replace(cp, needs_layout_passes=False)
    kern = pl.kernel(
        _edge_body,
        compiler_params=cp,
        out_type=(
            jax.ShapeDtypeStruct((2, ENT_PAD, CHANNEL), jnp.float32),
            jax.ShapeDtypeStruct((NUM_TILES, ENT_PAD), jnp.float32),
        ),
        mesh=mesh,
        scratch_types=[
            pltpu.VMEM((2, ECHUNK), jnp.int32),
            pltpu.VMEM((ECHUNK,), jnp.int32),
            pltpu.VMEM((ECHUNK, CHANNEL), jnp.float32),
            pltpu.VMEM((N_REL, CHANNEL), jnp.float32),
            pltpu.VMEM((ENT_PAD,), jnp.float32),
            pltpu.VMEM_SHARED((ENT_PAD, CHANNEL), jnp.float32),
        ],
    )
    return kern(entity_emb, weight, edata, head_p)


def _combine_body(psum_ref, pcnt_ref, out_ref):
    ssum = psum_ref[0] + psum_ref[1]
    cnt_t = jnp.transpose(pcnt_ref[...])
    cnt = jnp.sum(cnt_t, axis=1, keepdims=True)
    out_ref[...] = ssum / jnp.clip(cnt, 1.0, None)


def _combine(psum, pcnt):
    return pl.pallas_call(
        _combine_body,
        out_shape=jax.ShapeDtypeStruct((ENT_PAD, CHANNEL), jnp.float32),
    )(psum, pcnt)


def _user_body(im_ref, ent_ref, ue_ref, lat_ref, dwa_ref, w_ref, out_ref):
    mm = jnp.dot(im_ref[...], ent_ref[...], preferred_element_type=jnp.float32)
    score_ = lax.dot_general(ue_ref[...], lat_ref[...],
                             (((1,), (1,)), ((), ())),
                             preferred_element_type=jnp.float32)
    score = jax.nn.softmax(score_, axis=-1)
    dw = jnp.dot(jax.nn.softmax(dwa_ref[...], axis=-1), w_ref[...],
                 preferred_element_type=jnp.float32)
    gate = jnp.dot(score, dw, preferred_element_type=jnp.float32)
    out_ref[...] = mm * (1.0 + gate)


def _user_aggregate(interact_mat, entity_emb, user_emb, latent_emb,
                    disen_weight_att, weight):
    ub = 256
    grid = (N_USERS // ub,)
    return pl.pallas_call(
        _user_body,
        grid=grid,
        in_specs=[
            pl.BlockSpec((ub, N_ENTITIES), lambda i: (i, 0)),
            pl.BlockSpec((N_ENTITIES, CHANNEL), lambda i: (0, 0)),
            pl.BlockSpec((ub, CHANNEL), lambda i: (i, 0)),
            pl.BlockSpec((4, CHANNEL), lambda i: (0, 0)),
            pl.BlockSpec((4, N_REL), lambda i: (0, 0)),
            pl.BlockSpec((N_REL, CHANNEL), lambda i: (0, 0)),
        ],
        out_specs=pl.BlockSpec((ub, CHANNEL), lambda i: (i, 0)),
        out_shape=jax.ShapeDtypeStruct((N_USERS, CHANNEL), jnp.float32),
    )(interact_mat, entity_emb, user_emb, latent_emb, disen_weight_att, weight)


def kernel(entity_emb, user_emb, latent_emb, edge_index, edge_type,
           interact_mat, weight, disen_weight_att):
    head = edge_index[0].astype(jnp.int32)
    tail = edge_index[1].astype(jnp.int32)
    rel = (edge_type - 1).astype(jnp.int32)

    pad = EDGES_PAD - N_EDGES
    head_p = jnp.concatenate([head, jnp.full((pad,), N_ENTITIES, jnp.int32)])
    tail_p = jnp.concatenate([tail, jnp.zeros((pad,), jnp.int32)])
    rel_p = jnp.concatenate([rel, jnp.zeros((pad,), jnp.int32)])
    edata = jnp.stack([tail_p, rel_p]).reshape(2, TOTAL_CHUNKS, ECHUNK)
    edata = jnp.transpose(edata, (1, 0, 2))

    psum, pcnt = _edge_aggregate(entity_emb, weight, edata, head_p)
    entity_agg = _combine(psum, pcnt)[:N_ENTITIES]
    user_agg = _user_aggregate(interact_mat, entity_emb, user_emb, latent_emb,
                               disen_weight_att, weight)
    return (entity_agg, user_agg)


# speedup vs baseline: 4.5131x; 1.5420x over previous
import dataclasses

import jax
import jax.numpy as jnp
from jax import lax
from jax.experimental import pallas as pl
from jax.experimental.pallas import tpu as pltpu
from jax.experimental.pallas import tpu_sc as plsc

N_ENTITIES = 10000
CHANNEL = 128
N_EDGES = 320000
N_USERS = 4096
N_REL = 32

NUM_TILES = 32
ENT_PAD = 10112
ROWS_PER_TILE = ENT_PAD // 16
EDGES_PER_TILE = 10240
EDGES_PAD = EDGES_PER_TILE * NUM_TILES
ECHUNK = 128
CHUNKS_PER_TILE = EDGES_PER_TILE // ECHUNK
TOTAL_CHUNKS = EDGES_PAD // ECHUNK
RFULL = (ROWS_PER_TILE // ECHUNK) * ECHUNK
RTAIL = ROWS_PER_TILE - RFULL


def _edge_body(ent_hbm, w_hbm, edata_hbm, psum_hbm, pcnt_hbm,
               edata_v0, edata_v1, head_s0, head_s1, rows_v0, rows_v1,
               w_local, hist_v, acc, es0, es1, gs0, gs1, ss0, ss1):
    c = lax.axis_index("c")
    s = lax.axis_index("s")
    wid = s * 2 + c
    ones16 = jnp.ones((16,), jnp.float32)
    zeros16 = jnp.zeros((16,), jnp.float32)
    edata_v = (edata_v0, edata_v1)
    head_s = (head_s0, head_s1)
    rows_v = (rows_v0, rows_v1)
    esem = (es0, es1)
    gsem = (gs0, gs1)
    ssem = (ss0, ss1)

    pltpu.sync_copy(w_hbm, w_local)

    @pl.loop(0, ECHUNK)
    def _(i):
        for j in range(8):
            rows_v0[i, pl.ds(j * 16, 16)] = zeros16

    @pl.loop(0, ENT_PAD, step=16)
    def _(i):
        hist_v[pl.ds(i, 16)] = zeros16

    rbase = s * ROWS_PER_TILE

    @pl.loop(0, RFULL, step=ECHUNK)
    def _(k):
        pltpu.sync_copy(rows_v0, acc.at[pl.ds(rbase + k, ECHUNK)])

    pltpu.sync_copy(rows_v0.at[pl.ds(0, RTAIL)],
                    acc.at[pl.ds(rbase + RFULL, RTAIL)])

    cbase = wid * CHUNKS_PER_TILE
    last = CHUNKS_PER_TILE - 1

    pltpu.sync_copy(edata_hbm.at[cbase], edata_v0)
    pltpu.async_copy(edata_hbm.at[cbase + 1], edata_v1, es1)
    pltpu.async_copy(ent_hbm.at[edata_v0.at[0]], rows_v0, gs0)

    plsc.subcore_barrier()

    def _stage(g, b):
        o = 1 - b
        pltpu.make_async_copy(ent_hbm.at[edata_v[b].at[0]], rows_v[b],
                              gsem[b]).wait()
        pltpu.make_async_copy(edata_hbm.at[cbase], edata_v[o], esem[o]).wait()

        @pl.when(g > 0)
        def _():
            pltpu.make_async_copy(rows_v[o], acc.at[head_s[o]],
                                  ssem[o]).wait()

        pltpu.async_copy(ent_hbm.at[edata_v[o].at[0]], rows_v[o], gsem[o])

        @pl.loop(0, ECHUNK, step=16)
        def _(e):
            r16 = edata_v[b][1, pl.ds(e, 16)]
            h16 = edata_v[b][2, pl.ds(e, 16)]
            plsc.addupdate_scatter(hist_v, [h16], ones16)
            for k in range(16):
                r = r16[k]
                for j in range(8):
                    sl = pl.ds(j * 16, 16)
                    rows_v[b][e + k, sl] = rows_v[b][e + k, sl] * w_local[r, sl]

        for j in range(8):
            sl = pl.ds(j * 16, 16)
            head_s[b][sl] = edata_v[b][2, sl]

        pltpu.async_copy(rows_v[b], acc.at[head_s[b]], ssem[b], add=True)
        nxt = cbase + jnp.minimum(g + 2, last)
        pltpu.async_copy(edata_hbm.at[nxt], edata_v[b], esem[b])

    @pl.loop(0, CHUNKS_PER_TILE, step=2)
    def _(gg):
        _stage(gg, 0)
        _stage(gg + 1, 1)

    pltpu.make_async_copy(rows_v1, acc.at[head_s1], ss1).wait()
    pltpu.make_async_copy(ent_hbm.at[edata_v0.at[0]], rows_v0, gs0).wait()
    pltpu.make_async_copy(edata_hbm.at[cbase], edata_v1, es1).wait()

    plsc.subcore_barrier()

    @pl.loop(0, RFULL, step=ECHUNK)
    def _(k):
        pltpu.sync_copy(acc.at[pl.ds(rbase + k, ECHUNK)], rows_v0)
        pltpu.sync_copy(rows_v0, psum_hbm.at[c, pl.ds(rbase + k, ECHUNK)])

    pltpu.sync_copy(acc.at[pl.ds(rbase + RFULL, RTAIL)],
                    rows_v1.at[pl.ds(0, RTAIL)])
    pltpu.sync_copy(rows_v1.at[pl.ds(0, RTAIL)],
                    psum_hbm.at[c, pl.ds(rbase + RFULL, RTAIL)])

    pltpu.sync_copy(hist_v, pcnt_hbm.at[wid])


def _edge_aggregate(entity_emb, weight, edata):
    mesh = plsc.VectorSubcoreMesh(core_axis_name="c", subcore_axis_name="s")
    cp = pltpu.CompilerParams()
    if "needs_layout_passes" in pltpu.CompilerParams.__dataclass_fields__:
        cp = dataclasses.replace(cp, needs_layout_passes=False)
    kern = pl.kernel(
        _edge_body,
        compiler_params=cp,
        out_type=(
            jax.ShapeDtypeStruct((2, ENT_PAD, CHANNEL), jnp.float32),
            jax.ShapeDtypeStruct((NUM_TILES, ENT_PAD), jnp.float32),
        ),
        mesh=mesh,
        scratch_types=[
            pltpu.VMEM((3, ECHUNK), jnp.int32),
            pltpu.VMEM((3, ECHUNK), jnp.int32),
            pltpu.VMEM((ECHUNK,), jnp.int32),
            pltpu.VMEM((ECHUNK,), jnp.int32),
            pltpu.VMEM((ECHUNK, CHANNEL), jnp.float32),
            pltpu.VMEM((ECHUNK, CHANNEL), jnp.float32),
            pltpu.VMEM((N_REL, CHANNEL), jnp.float32),
            pltpu.VMEM((ENT_PAD,), jnp.float32),
            pltpu.VMEM_SHARED((ENT_PAD, CHANNEL), jnp.float32),
            pltpu.SemaphoreType.DMA,
            pltpu.SemaphoreType.DMA,
            pltpu.SemaphoreType.DMA,
            pltpu.SemaphoreType.DMA,
            pltpu.SemaphoreType.DMA,
            pltpu.SemaphoreType.DMA,
        ],
    )
    return kern(entity_emb, weight, edata)


def _combine_body(psum_ref, pcnt_ref, out_ref):
    ssum = psum_ref[0] + psum_ref[1]
    cnt_t = jnp.transpose(pcnt_ref[...])
    cnt = jnp.sum(cnt_t, axis=1, keepdims=True)
    out_ref[...] = ssum / jnp.clip(cnt, 1.0, None)


def _combine(psum, pcnt):
    return pl.pallas_call(
        _combine_body,
        out_shape=jax.ShapeDtypeStruct((ENT_PAD, CHANNEL), jnp.float32),
    )(psum, pcnt)


def _user_body(im_ref, ent_ref, ue_ref, lat_ref, dwa_ref, w_ref, out_ref):
    mm = jnp.dot(im_ref[...], ent_ref[...], preferred_element_type=jnp.float32)
    score_ = lax.dot_general(ue_ref[...], lat_ref[...],
                             (((1,), (1,)), ((), ())),
                             preferred_element_type=jnp.float32)
    score = jax.nn.softmax(score_, axis=-1)
    dw = jnp.dot(jax.nn.softmax(dwa_ref[...], axis=-1), w_ref[...],
                 preferred_element_type=jnp.float32)
    gate = jnp.dot(score, dw, preferred_element_type=jnp.float32)
    out_ref[...] = mm * (1.0 + gate)


def _user_aggregate(interact_mat, entity_emb, user_emb, latent_emb,
                    disen_weight_att, weight):
    ub = 256
    grid = (N_USERS // ub,)
    return pl.pallas_call(
        _user_body,
        grid=grid,
        in_specs=[
            pl.BlockSpec((ub, N_ENTITIES), lambda i: (i, 0)),
            pl.BlockSpec((N_ENTITIES, CHANNEL), lambda i: (0, 0)),
            pl.BlockSpec((ub, CHANNEL), lambda i: (i, 0)),
            pl.BlockSpec((4, CHANNEL), lambda i: (0, 0)),
            pl.BlockSpec((4, N_REL), lambda i: (0, 0)),
            pl.BlockSpec((N_REL, CHANNEL), lambda i: (0, 0)),
        ],
        out_specs=pl.BlockSpec((ub, CHANNEL), lambda i: (i, 0)),
        out_shape=jax.ShapeDtypeStruct((N_USERS, CHANNEL), jnp.float32),
    )(interact_mat, entity_emb, user_emb, latent_emb, disen_weight_att, weight)


def kernel(entity_emb, user_emb, latent_emb, edge_index, edge_type,
           interact_mat, weight, disen_weight_att):
    head = edge_index[0].astype(jnp.int32)
    tail = edge_index[1].astype(jnp.int32)
    rel = (edge_type - 1).astype(jnp.int32)

    pad = EDGES_PAD - N_EDGES
    head_p = jnp.concatenate([head, jnp.full((pad,), N_ENTITIES, jnp.int32)])
    tail_p = jnp.concatenate([tail, jnp.zeros((pad,), jnp.int32)])
    rel_p = jnp.concatenate([rel, jnp.zeros((pad,), jnp.int32)])
    edata = jnp.stack([tail_p, rel_p, head_p]).reshape(3, TOTAL_CHUNKS, ECHUNK)
    edata = jnp.transpose(edata, (1, 0, 2))

    psum, pcnt = _edge_aggregate(entity_emb, weight, edata)
    entity_agg = _combine(psum, pcnt)[:N_ENTITIES]
    user_agg = _user_aggregate(interact_mat, entity_emb, user_emb, latent_emb,
                               disen_weight_att, weight)
    return (entity_agg, user_agg)


# speedup vs baseline: 4.5879x; 1.0166x over previous
import dataclasses

import jax
import jax.numpy as jnp
from jax import lax
from jax.experimental import pallas as pl
from jax.experimental.pallas import tpu as pltpu
from jax.experimental.pallas import tpu_sc as plsc

N_ENTITIES = 10000
CHANNEL = 128
N_EDGES = 320000
N_USERS = 4096
N_REL = 32

NUM_TILES = 32
ENT_PAD = 10112
ROWS_PER_TILE = ENT_PAD // 16
EDGES_PER_TILE = 10240
EDGES_PAD = EDGES_PER_TILE * NUM_TILES
ECHUNK = 128
CHUNKS_PER_TILE = EDGES_PER_TILE // ECHUNK
TOTAL_CHUNKS = EDGES_PAD // ECHUNK
RFULL = (ROWS_PER_TILE // ECHUNK) * ECHUNK
RTAIL = ROWS_PER_TILE - RFULL


def _edge_body(ent_hbm, w_hbm, edata_hbm, psum_hbm, pcnt_hbm,
               edata_v0, edata_v1, head_s0, head_s1, rows_v0, rows_v1,
               w_local, hist_v, acc, es0, es1, gs0, gs1, ss0, ss1):
    c = lax.axis_index("c")
    s = lax.axis_index("s")
    wid = s * 2 + c
    ones16 = jnp.ones((16,), jnp.float32)
    zeros16 = jnp.zeros((16,), jnp.float32)
    edata_v = (edata_v0, edata_v1)
    head_s = (head_s0, head_s1)
    rows_v = (rows_v0, rows_v1)
    esem = (es0, es1)
    gsem = (gs0, gs1)
    ssem = (ss0, ss1)

    pltpu.sync_copy(w_hbm, w_local)

    @pl.loop(0, ECHUNK)
    def _(i):
        for j in range(8):
            rows_v0[i, pl.ds(j * 16, 16)] = zeros16

    @pl.loop(0, ENT_PAD, step=16)
    def _(i):
        hist_v[pl.ds(i, 16)] = zeros16

    rbase = s * ROWS_PER_TILE

    @pl.loop(0, RFULL, step=ECHUNK)
    def _(k):
        pltpu.sync_copy(rows_v0, acc.at[pl.ds(rbase + k, ECHUNK)])

    pltpu.sync_copy(rows_v0.at[pl.ds(0, RTAIL)],
                    acc.at[pl.ds(rbase + RFULL, RTAIL)])

    cbase = wid * CHUNKS_PER_TILE
    last = CHUNKS_PER_TILE - 1

    pltpu.sync_copy(edata_hbm.at[cbase], edata_v0)
    pltpu.async_copy(edata_hbm.at[cbase + 1], edata_v1, es1)
    pltpu.async_copy(ent_hbm.at[edata_v0.at[0]], rows_v0, gs0)

    plsc.subcore_barrier()

    def _stage(g, b):
        o = 1 - b
        pltpu.make_async_copy(ent_hbm.at[edata_v[b].at[0]], rows_v[b],
                              gsem[b]).wait()
        pltpu.make_async_copy(edata_hbm.at[cbase], edata_v[o], esem[o]).wait()

        @pl.when(g > 0)
        def _():
            pltpu.make_async_copy(rows_v[o], acc.at[head_s[o]],
                                  ssem[o]).wait()

        pltpu.async_copy(ent_hbm.at[edata_v[o].at[0]], rows_v[o], gsem[o])

        @pl.loop(0, ECHUNK, step=16)
        def _(e):
            r16 = edata_v[b][1, pl.ds(e, 16)]
            h16 = edata_v[b][2, pl.ds(e, 16)]
            plsc.addupdate_scatter(hist_v, [h16], ones16)
            for k in range(16):
                r = r16[k]
                row = [rows_v[b][e + k, pl.ds(j * 16, 16)] for j in range(8)]
                wrow = [w_local[r, pl.ds(j * 16, 16)] for j in range(8)]
                for j in range(8):
                    rows_v[b][e + k, pl.ds(j * 16, 16)] = row[j] * wrow[j]

        for j in range(8):
            sl = pl.ds(j * 16, 16)
            head_s[b][sl] = edata_v[b][2, sl]

        pltpu.async_copy(rows_v[b], acc.at[head_s[b]], ssem[b], add=True)
        nxt = cbase + jnp.minimum(g + 2, last)
        pltpu.async_copy(edata_hbm.at[nxt], edata_v[b], esem[b])

    @pl.loop(0, CHUNKS_PER_TILE, step=2)
    def _(gg):
        _stage(gg, 0)
        _stage(gg + 1, 1)

    pltpu.make_async_copy(rows_v1, acc.at[head_s1], ss1).wait()
    pltpu.make_async_copy(ent_hbm.at[edata_v0.at[0]], rows_v0, gs0).wait()
    pltpu.make_async_copy(edata_hbm.at[cbase], edata_v1, es1).wait()

    plsc.subcore_barrier()

    @pl.loop(0, RFULL, step=ECHUNK)
    def _(k):
        pltpu.sync_copy(acc.at[pl.ds(rbase + k, ECHUNK)], rows_v0)
        pltpu.sync_copy(rows_v0, psum_hbm.at[c, pl.ds(rbase + k, ECHUNK)])

    pltpu.sync_copy(acc.at[pl.ds(rbase + RFULL, RTAIL)],
                    rows_v1.at[pl.ds(0, RTAIL)])
    pltpu.sync_copy(rows_v1.at[pl.ds(0, RTAIL)],
                    psum_hbm.at[c, pl.ds(rbase + RFULL, RTAIL)])

    pltpu.sync_copy(hist_v, pcnt_hbm.at[wid])


def _edge_aggregate(entity_emb, weight, edata):
    mesh = plsc.VectorSubcoreMesh(core_axis_name="c", subcore_axis_name="s")
    cp = pltpu.CompilerParams()
    if "needs_layout_passes" in pltpu.CompilerParams.__dataclass_fields__:
        cp = dataclasses.replace(cp, needs_layout_passes=False)
    kern = pl.kernel(
        _edge_body,
        compiler_params=cp,
        out_type=(
            jax.ShapeDtypeStruct((2, ENT_PAD, CHANNEL), jnp.float32),
            jax.ShapeDtypeStruct((NUM_TILES, ENT_PAD), jnp.float32),
        ),
        mesh=mesh,
        scratch_types=[
            pltpu.VMEM((3, ECHUNK), jnp.int32),
            pltpu.VMEM((3, ECHUNK), jnp.int32),
            pltpu.VMEM((ECHUNK,), jnp.int32),
            pltpu.VMEM((ECHUNK,), jnp.int32),
            pltpu.VMEM((ECHUNK, CHANNEL), jnp.float32),
            pltpu.VMEM((ECHUNK, CHANNEL), jnp.float32),
            pltpu.VMEM((N_REL, CHANNEL), jnp.float32),
            pltpu.VMEM((ENT_PAD,), jnp.float32),
            pltpu.VMEM_SHARED((ENT_PAD, CHANNEL), jnp.float32),
            pltpu.SemaphoreType.DMA,
            pltpu.SemaphoreType.DMA,
            pltpu.SemaphoreType.DMA,
            pltpu.SemaphoreType.DMA,
            pltpu.SemaphoreType.DMA,
            pltpu.SemaphoreType.DMA,
        ],
    )
    return kern(entity_emb, weight, edata)


def _combine_body(psum_ref, pcnt_ref, out_ref):
    ssum = psum_ref[0] + psum_ref[1]
    cnt_t = jnp.transpose(pcnt_ref[...])
    cnt = jnp.sum(cnt_t, axis=1, keepdims=True)
    out_ref[...] = ssum / jnp.clip(cnt, 1.0, None)


def _combine(psum, pcnt):
    return pl.pallas_call(
        _combine_body,
        out_shape=jax.ShapeDtypeStruct((ENT_PAD, CHANNEL), jnp.float32),
    )(psum, pcnt)


def _user_body(im_ref, ent_ref, ue_ref, lat_ref, dwa_ref, w_ref, out_ref):
    mm = jnp.dot(im_ref[...], ent_ref[...], preferred_element_type=jnp.float32)
    score_ = lax.dot_general(ue_ref[...], lat_ref[...],
                             (((1,), (1,)), ((), ())),
                             preferred_element_type=jnp.float32)
    score = jax.nn.softmax(score_, axis=-1)
    dw = jnp.dot(jax.nn.softmax(dwa_ref[...], axis=-1), w_ref[...],
                 preferred_element_type=jnp.float32)
    gate = jnp.dot(score, dw, preferred_element_type=jnp.float32)
    out_ref[...] = mm * (1.0 + gate)


def _user_aggregate(interact_mat, entity_emb, user_emb, latent_emb,
                    disen_weight_att, weight):
    ub = 256
    grid = (N_USERS // ub,)
    return pl.pallas_call(
        _user_body,
        grid=grid,
        in_specs=[
            pl.BlockSpec((ub, N_ENTITIES), lambda i: (i, 0)),
            pl.BlockSpec((N_ENTITIES, CHANNEL), lambda i: (0, 0)),
            pl.BlockSpec((ub, CHANNEL), lambda i: (i, 0)),
            pl.BlockSpec((4, CHANNEL), lambda i: (0, 0)),
            pl.BlockSpec((4, N_REL), lambda i: (0, 0)),
            pl.BlockSpec((N_REL, CHANNEL), lambda i: (0, 0)),
        ],
        out_specs=pl.BlockSpec((ub, CHANNEL), lambda i: (i, 0)),
        out_shape=jax.ShapeDtypeStruct((N_USERS, CHANNEL), jnp.float32),
    )(interact_mat, entity_emb, user_emb, latent_emb, disen_weight_att, weight)


def kernel(entity_emb, user_emb, latent_emb, edge_index, edge_type,
           interact_mat, weight, disen_weight_att):
    head = edge_index[0].astype(jnp.int32)
    tail = edge_index[1].astype(jnp.int32)
    rel = (edge_type - 1).astype(jnp.int32)

    pad = EDGES_PAD - N_EDGES
    head_p = jnp.concatenate([head, jnp.full((pad,), N_ENTITIES, jnp.int32)])
    tail_p = jnp.concatenate([tail, jnp.zeros((pad,), jnp.int32)])
    rel_p = jnp.concatenate([rel, jnp.zeros((pad,), jnp.int32)])
    edata = jnp.stack([tail_p, rel_p, head_p]).reshape(3, TOTAL_CHUNKS, ECHUNK)
    edata = jnp.transpose(edata, (1, 0, 2))

    psum, pcnt = _edge_aggregate(entity_emb, weight, edata)
    entity_agg = _combine(psum, pcnt)[:N_ENTITIES]
    user_agg = _user_aggregate(interact_mat, entity_emb, user_emb, latent_emb,
                               disen_weight_att, weight)
    return (entity_agg, user_agg)


# speedup vs baseline: 4.5888x; 1.0002x over previous
import dataclasses

import jax
import jax.numpy as jnp
from jax import lax
from jax.experimental import pallas as pl
from jax.experimental.pallas import tpu as pltpu
from jax.experimental.pallas import tpu_sc as plsc

N_ENTITIES = 10000
CHANNEL = 128
N_EDGES = 320000
N_USERS = 4096
N_REL = 32

NUM_TILES = 32
ENT_PAD = 10112
ROWS_PER_TILE = ENT_PAD // 16
EDGES_PER_TILE = 10240
EDGES_PAD = EDGES_PER_TILE * NUM_TILES
ECHUNK = 128
CHUNKS_PER_TILE = EDGES_PER_TILE // ECHUNK
TOTAL_CHUNKS = EDGES_PAD // ECHUNK
RFULL = (ROWS_PER_TILE // ECHUNK) * ECHUNK
RTAIL = ROWS_PER_TILE - RFULL


def _edge_body(ent_hbm, w_hbm, edata_hbm, psum_hbm, pcnt_hbm,
               edata_v0, edata_v1, head_s0, head_s1, rows_v0, rows_v1,
               w_local, hist_v, acc, es0, es1, gs0, gs1, ss0, ss1,
               gh0, gh1, sh0, sh1):
    c = lax.axis_index("c")
    s = lax.axis_index("s")
    wid = s * 2 + c
    ones16 = jnp.ones((16,), jnp.float32)
    zeros16 = jnp.zeros((16,), jnp.float32)
    edata_v = (edata_v0, edata_v1)
    head_s = (head_s0, head_s1)
    rows_v = (rows_v0, rows_v1)
    esem = (es0, es1)
    gsem = (gs0, gs1)
    ssem = (ss0, ss1)
    gsemh = (gh0, gh1)
    ssemh = (sh0, sh1)
    H = ECHUNK // 2

    def _gather_lo(b, sem):
        return pltpu.make_async_copy(
            ent_hbm.at[edata_v[b].at[0, pl.ds(0, H)]],
            rows_v[b].at[pl.ds(0, H)], sem)

    def _gather_hi(b, sem):
        return pltpu.make_async_copy(
            ent_hbm.at[edata_v[b].at[0, pl.ds(H, H)]],
            rows_v[b].at[pl.ds(H, H)], sem)

    pltpu.sync_copy(w_hbm, w_local)

    @pl.loop(0, ECHUNK)
    def _(i):
        for j in range(8):
            rows_v0[i, pl.ds(j * 16, 16)] = zeros16

    @pl.loop(0, ENT_PAD, step=16)
    def _(i):
        hist_v[pl.ds(i, 16)] = zeros16

    rbase = s * ROWS_PER_TILE

    @pl.loop(0, RFULL, step=ECHUNK)
    def _(k):
        pltpu.sync_copy(rows_v0, acc.at[pl.ds(rbase + k, ECHUNK)])

    pltpu.sync_copy(rows_v0.at[pl.ds(0, RTAIL)],
                    acc.at[pl.ds(rbase + RFULL, RTAIL)])

    cbase = wid * CHUNKS_PER_TILE
    last = CHUNKS_PER_TILE - 1

    pltpu.sync_copy(edata_hbm.at[cbase], edata_v0)
    pltpu.async_copy(edata_hbm.at[cbase + 1], edata_v1, es1)
    _gather_lo(0, gs0).start()
    _gather_hi(0, gh0).start()

    plsc.subcore_barrier()

    def _stage(g, b):
        o = 1 - b
        _gather_lo(b, gsem[b]).wait()
        _gather_hi(b, gsemh[b]).wait()
        pltpu.make_async_copy(edata_hbm.at[cbase], edata_v[o], esem[o]).wait()

        @pl.when(g > 0)
        def _():
            pltpu.make_async_copy(rows_v[o].at[pl.ds(0, H)],
                                  acc.at[head_s[o].at[0]], ssem[o]).wait()
            pltpu.make_async_copy(rows_v[o].at[pl.ds(H, H)],
                                  acc.at[head_s[o].at[1]], ssemh[o]).wait()

        _gather_lo(o, gsem[o]).start()
        _gather_hi(o, gsemh[o]).start()

        @pl.loop(0, ECHUNK, step=16)
        def _(e):
            r16 = edata_v[b][1, pl.ds(e, 16)]
            h16 = edata_v[b][2, pl.ds(e, 16)]
            plsc.addupdate_scatter(hist_v, [h16], ones16)
            for k in range(16):
                r = r16[k]
                row = [rows_v[b][e + k, pl.ds(j * 16, 16)] for j in range(8)]
                wrow = [w_local[r, pl.ds(j * 16, 16)] for j in range(8)]
                for j in range(8):
                    rows_v[b][e + k, pl.ds(j * 16, 16)] = row[j] * wrow[j]

        for j in range(4):
            sl = pl.ds(j * 16, 16)
            head_s[b][0, sl] = edata_v[b][2, sl]
        for j in range(4):
            sl = pl.ds(j * 16, 16)
            head_s[b][1, sl] = edata_v[b][2, pl.ds(H + j * 16, 16)]

        pltpu.async_copy(rows_v[b].at[pl.ds(0, H)], acc.at[head_s[b].at[0]],
                         ssem[b], add=True)
        pltpu.async_copy(rows_v[b].at[pl.ds(H, H)], acc.at[head_s[b].at[1]],
                         ssemh[b], add=True)
        nxt = cbase + jnp.minimum(g + 2, last)
        pltpu.async_copy(edata_hbm.at[nxt], edata_v[b], esem[b])

    @pl.loop(0, CHUNKS_PER_TILE, step=2)
    def _(gg):
        _stage(gg, 0)
        _stage(gg + 1, 1)

    pltpu.make_async_copy(rows_v1.at[pl.ds(0, H)], acc.at[head_s1.at[0]],
                          ss1).wait()
    pltpu.make_async_copy(rows_v1.at[pl.ds(H, H)], acc.at[head_s1.at[1]],
                          sh1).wait()
    _gather_lo(0, gs0).wait()
    _gather_hi(0, gh0).wait()
    pltpu.make_async_copy(edata_hbm.at[cbase], edata_v1, es1).wait()

    plsc.subcore_barrier()

    @pl.loop(0, RFULL, step=ECHUNK)
    def _(k):
        pltpu.sync_copy(acc.at[pl.ds(rbase + k, ECHUNK)], rows_v0)
        pltpu.sync_copy(rows_v0, psum_hbm.at[c, pl.ds(rbase + k, ECHUNK)])

    pltpu.sync_copy(acc.at[pl.ds(rbase + RFULL, RTAIL)],
                    rows_v1.at[pl.ds(0, RTAIL)])
    pltpu.sync_copy(rows_v1.at[pl.ds(0, RTAIL)],
                    psum_hbm.at[c, pl.ds(rbase + RFULL, RTAIL)])

    pltpu.sync_copy(hist_v, pcnt_hbm.at[wid])


def _edge_aggregate(entity_emb, weight, edata):
    mesh = plsc.VectorSubcoreMesh(core_axis_name="c", subcore_axis_name="s")
    cp = pltpu.CompilerParams()
    if "needs_layout_passes" in pltpu.CompilerParams.__dataclass_fields__:
        cp = dataclasses.replace(cp, needs_layout_passes=False)
    kern = pl.kernel(
        _edge_body,
        compiler_params=cp,
        out_type=(
            jax.ShapeDtypeStruct((2, ENT_PAD, CHANNEL), jnp.float32),
            jax.ShapeDtypeStruct((NUM_TILES, ENT_PAD), jnp.float32),
        ),
        mesh=mesh,
        scratch_types=[
            pltpu.VMEM((3, ECHUNK), jnp.int32),
            pltpu.VMEM((3, ECHUNK), jnp.int32),
            pltpu.VMEM((2, ECHUNK // 2), jnp.int32),
            pltpu.VMEM((2, ECHUNK // 2), jnp.int32),
            pltpu.VMEM((ECHUNK, CHANNEL), jnp.float32),
            pltpu.VMEM((ECHUNK, CHANNEL), jnp.float32),
            pltpu.VMEM((N_REL, CHANNEL), jnp.float32),
            pltpu.VMEM((ENT_PAD,), jnp.float32),
            pltpu.VMEM_SHARED((ENT_PAD, CHANNEL), jnp.float32),
            pltpu.SemaphoreType.DMA,
            pltpu.SemaphoreType.DMA,
            pltpu.SemaphoreType.DMA,
            pltpu.SemaphoreType.DMA,
            pltpu.SemaphoreType.DMA,
            pltpu.SemaphoreType.DMA,
            pltpu.SemaphoreType.DMA,
            pltpu.SemaphoreType.DMA,
            pltpu.SemaphoreType.DMA,
            pltpu.SemaphoreType.DMA,
        ],
    )
    return kern(entity_emb, weight, edata)


def _combine_body(psum_ref, pcnt_ref, out_ref):
    ssum = psum_ref[0] + psum_ref[1]
    cnt_t = jnp.transpose(pcnt_ref[...])
    cnt = jnp.sum(cnt_t, axis=1, keepdims=True)
    out_ref[...] = ssum / jnp.clip(cnt, 1.0, None)


def _combine(psum, pcnt):
    return pl.pallas_call(
        _combine_body,
        out_shape=jax.ShapeDtypeStruct((ENT_PAD, CHANNEL), jnp.float32),
    )(psum, pcnt)


def _user_body(im_ref, ent_ref, ue_ref, lat_ref, dwa_ref, w_ref, out_ref):
    mm = jnp.dot(im_ref[...], ent_ref[...], preferred_element_type=jnp.float32)
    score_ = lax.dot_general(ue_ref[...], lat_ref[...],
                             (((1,), (1,)), ((), ())),
                             preferred_element_type=jnp.float32)
    score = jax.nn.softmax(score_, axis=-1)
    dw = jnp.dot(jax.nn.softmax(dwa_ref[...], axis=-1), w_ref[...],
                 preferred_element_type=jnp.float32)
    gate = jnp.dot(score, dw, preferred_element_type=jnp.float32)
    out_ref[...] = mm * (1.0 + gate)


def _user_aggregate(interact_mat, entity_emb, user_emb, latent_emb,
                    disen_weight_att, weight):
    ub = 256
    grid = (N_USERS // ub,)
    return pl.pallas_call(
        _user_body,
        grid=grid,
        in_specs=[
            pl.BlockSpec((ub, N_ENTITIES), lambda i: (i, 0)),
            pl.BlockSpec((N_ENTITIES, CHANNEL), lambda i: (0, 0)),
            pl.BlockSpec((ub, CHANNEL), lambda i: (i, 0)),
            pl.BlockSpec((4, CHANNEL), lambda i: (0, 0)),
            pl.BlockSpec((4, N_REL), lambda i: (0, 0)),
            pl.BlockSpec((N_REL, CHANNEL), lambda i: (0, 0)),
        ],
        out_specs=pl.BlockSpec((ub, CHANNEL), lambda i: (i, 0)),
        out_shape=jax.ShapeDtypeStruct((N_USERS, CHANNEL), jnp.float32),
    )(interact_mat, entity_emb, user_emb, latent_emb, disen_weight_att, weight)


def kernel(entity_emb, user_emb, latent_emb, edge_index, edge_type,
           interact_mat, weight, disen_weight_att):
    head = edge_index[0].astype(jnp.int32)
    tail = edge_index[1].astype(jnp.int32)
    rel = (edge_type - 1).astype(jnp.int32)

    pad = EDGES_PAD - N_EDGES
    head_p = jnp.concatenate([head, jnp.full((pad,), N_ENTITIES, jnp.int32)])
    tail_p = jnp.concatenate([tail, jnp.zeros((pad,), jnp.int32)])
    rel_p = jnp.concatenate([rel, jnp.zeros((pad,), jnp.int32)])
    edata = jnp.stack([tail_p, rel_p, head_p]).reshape(3, TOTAL_CHUNKS, ECHUNK)
    edata = jnp.transpose(edata, (1, 0, 2))

    psum, pcnt = _edge_aggregate(entity_emb, weight, edata)
    entity_agg = _combine(psum, pcnt)[:N_ENTITIES]
    user_agg = _user_aggregate(interact_mat, entity_emb, user_emb, latent_emb,
                               disen_weight_att, weight)
    return (entity_agg, user_agg)
